# baseline (speedup 1.0000x reference)
import functools

import numpy as np
import jax
import jax.numpy as jnp
from jax import lax
from jax.experimental import pallas as pl
from jax.experimental.pallas import tpu as pltpu
from jax.experimental.pallas import tpu_sc as plsc

_B = 8


def _interp_mat(h):
    ys = np.linspace(0.0, h - 1, 2 * h, dtype=np.float32)
    y0 = np.floor(ys).astype(np.int32)
    y1 = np.minimum(y0 + 1, h - 1)
    wy = (ys - y0).astype(np.float32)
    m = np.zeros((2 * h, h), dtype=np.float32)
    rows = np.arange(2 * h)
    m[rows, y0] += 1.0 - wy
    m[rows, y1] += wy
    return m


def _lane_gather(x2, idx, cin, cout):
    r = x2.shape[0]
    idxb = jnp.broadcast_to((idx & 127)[None, :], (r, cout))
    hi = idx >> 7
    acc = jnp.take_along_axis(x2[:, 0:128], idxb, axis=1)
    for h in range(1, cin // 128):
        g = jnp.take_along_axis(x2[:, h * 128:(h + 1) * 128], idxb, axis=1)
        sel = jnp.broadcast_to((hi == h)[None, :], (r, cout))
        acc = jnp.where(sel, g, acc)
    return acc


def _gather2_only(p2, i12):
    def body(x_ref, i_ref, g_ref):
        x2 = x_ref[...].reshape(28 * 8, 512)
        g_ref[...] = _lane_gather(x2, i_ref[...], 512, 512).reshape(1, 28, 8, 512)

    return pl.pallas_call(
        body,
        grid=(28,),
        in_specs=[
            pl.BlockSpec((1, 28, 8, 512), lambda h: (h, 0, 0, 0)),
            pl.BlockSpec((512,), lambda h: (0,)),
        ],
        out_specs=pl.BlockSpec((1, 28, 8, 512), lambda h: (h, 0, 0, 0)),
        out_shape=jax.ShapeDtypeStruct((28, 28, _B, 512), jnp.float32),
    )(p2, i12)


def _gather3(p3, i23):
    def body(x_ref, i_ref, o_ref):
        x2 = x_ref[...].reshape(14 * 8, 1024)
        g = _lane_gather(x2, i_ref[...], 1024, 512)
        o_ref[...] = g.reshape(1, 14, 8, 512)

    return pl.pallas_call(
        body,
        grid=(14,),
        in_specs=[
            pl.BlockSpec((1, 14, 8, 1024), lambda h: (h, 0, 0, 0)),
            pl.BlockSpec((512,), lambda h: (0,)),
        ],
        out_specs=pl.BlockSpec((1, 14, 8, 512), lambda h: (h, 0, 0, 0)),
        out_shape=jax.ShapeDtypeStruct((14, 14, _B, 512), jnp.float32),
    )(p3, i23)


def _merge1(o1c):
    def body(x_ref, o_ref):
        o_ref[...] = x_ref[...]

    return pl.pallas_call(
        body,
        grid=(_B,),
        in_specs=[pl.BlockSpec((1, 56, 56, 256), lambda b: (b, 0, 0, 0))],
        out_specs=pl.BlockSpec((1, 56, 56, 256), lambda b: (b, 0, 0, 0)),
        out_shape=jax.ShapeDtypeStruct((_B, 56, 56, 768), jnp.float32),
    )(o1c)


def _merge2(o2c):
    def body(x_ref, o_ref):
        o_ref[...] = x_ref[...]

    return pl.pallas_call(
        body,
        grid=(28,),
        in_specs=[pl.BlockSpec((1, 28, 8, 512), lambda h: (h, 0, 0, 0))],
        out_specs=pl.BlockSpec((1, 28, 8, 512), lambda h: (h, 0, 0, 0)),
        out_shape=jax.ShapeDtypeStruct((28, 28, _B, 1024), jnp.float32),
    )(o2c)


def _upsample1(g2, o1n_partial):
    m2 = jnp.asarray(_interp_mat(28))

    def body(o_alias_ref, top_ref, bot_ref, m_ref, o_ref):
        del o_alias_ref
        i = pl.program_id(1)
        wy = (((i * 27) % 55).astype(jnp.float32) / 55.0).astype(jnp.float32)
        u = top_ref[0] * (1.0 - wy) + bot_ref[0] * wy
        v = jax.lax.dot_general(m_ref[...], u, (((1,), (0,)), ((), ())),
                                preferred_element_type=jnp.float32)
        o_ref[...] = v.transpose(1, 0, 2).reshape(_B, 1, 56, 256)

    return pl.pallas_call(
        body,
        grid=(2, 56),
        in_specs=[
            pl.BlockSpec(memory_space=pl.ANY),
            pl.BlockSpec((1, 28, 8, 256), lambda cq, i: ((i * 27) // 55, 0, 0, cq)),
            pl.BlockSpec((1, 28, 8, 256),
                         lambda cq, i: (jnp.minimum((i * 27) // 55 + 1, 27), 0, 0, cq)),
            pl.BlockSpec((56, 28), lambda cq, i: (0, 0)),
        ],
        out_specs=pl.BlockSpec((_B, 1, 56, 256), lambda cq, i: (0, i, 0, 1 + cq)),
        out_shape=jax.ShapeDtypeStruct((_B, 56, 56, 768), jnp.float32),
        input_output_aliases={0: 0},
    )(o1n_partial, g2, g2, m2)


def _upsample2(g3, o2n_partial):
    m3 = jnp.asarray(_interp_mat(14))

    def body(o_alias_ref, top_ref, bot_ref, m_ref, o_ref):
        del o_alias_ref
        i = pl.program_id(0)
        wy = (((i * 13) % 27).astype(jnp.float32) / 27.0).astype(jnp.float32)
        u = top_ref[0] * (1.0 - wy) + bot_ref[0] * wy
        v = jax.lax.dot_general(m_ref[...], u, (((1,), (0,)), ((), ())),
                                preferred_element_type=jnp.float32)
        o_ref[...] = v.reshape(1, 28, _B, 512)

    return pl.pallas_call(
        body,
        grid=(28,),
        in_specs=[
            pl.BlockSpec(memory_space=pl.ANY),
            pl.BlockSpec((1, 14, 8, 512), lambda i: ((i * 13) // 27, 0, 0, 0)),
            pl.BlockSpec((1, 14, 8, 512),
                         lambda i: (jnp.minimum((i * 13) // 27 + 1, 13), 0, 0, 0)),
            pl.BlockSpec((28, 14), lambda i: (0, 0)),
        ],
        out_specs=pl.BlockSpec((1, 28, _B, 512), lambda i: (i, 0, 0, 1)),
        out_shape=jax.ShapeDtypeStruct((28, 28, _B, 1024), jnp.float32),
        input_output_aliases={0: 0},
    )(o2n_partial, g3, g3, m3)


def _sc_skip_fills(v1, v2, map1, map2):
    mesh = plsc.VectorSubcoreMesh(core_axis_name="c", subcore_axis_name="s")
    out_type = [
        jax.ShapeDtypeStruct((448 * 14336,), jnp.float32),
        jax.ShapeDtypeStruct((784 * 4096,), jnp.float32),
    ]
    scratch_types = [
        pltpu.VMEM((14336,), jnp.int32),
        pltpu.VMEM((4096,), jnp.int32),
        pltpu.VMEM((14336,), jnp.float32),
        pltpu.VMEM((14336,), jnp.float32),
        pltpu.VMEM((14336,), jnp.float32),
        pltpu.VMEM((14336,), jnp.float32),
        pltpu.VMEM((4096,), jnp.float32),
        pltpu.VMEM((4096,), jnp.float32),
        pltpu.VMEM((4096,), jnp.float32),
        pltpu.VMEM((4096,), jnp.float32),
        pltpu.SemaphoreType.DMA,
        pltpu.SemaphoreType.DMA,
        pltpu.SemaphoreType.DMA,
        pltpu.SemaphoreType.DMA,
    ]

    @functools.partial(pl.kernel, out_type=out_type, mesh=mesh,
                       scratch_types=scratch_types,
                       compiler_params=pltpu.CompilerParams(
                           needs_layout_passes=False))
    def k(v1h, v2h, m1h, m2h, o1h, o2h,
          m1v, m2v, in1a, in1b, ot1a, ot1b, in2a, in2b, ot2a, ot2b,
          sia, sib, soa, sob):
        wid = lax.axis_index("s") * 2 + lax.axis_index("c")
        pltpu.sync_copy(m1h, m1v)
        pltpu.sync_copy(m2h, m2v)

        ins1 = (in1a, in1b)
        ots1 = (ot1a, ot1b)
        sin = (sia, sib)
        sot = (soa, sob)

        def in1_cp(t, par):
            p = wid + 32 * t
            return pltpu.make_async_copy(
                v1h.at[pl.ds(p * 14336, 14336)], ins1[par], sin[par])

        def ot1_cp(t, par):
            p = wid + 32 * t
            return pltpu.make_async_copy(
                ots1[par], o1h.at[pl.ds(p * 14336, 14336)], sot[par])

        in1_cp(0, 0).start()
        for t in range(14):
            par = t % 2
            in1_cp(t, par).wait()
            if t + 1 < 14:
                in1_cp(t + 1, 1 - par).start()
            if t >= 2:
                ot1_cp(t - 2, par).wait()

            def c1(j, carry, _par=par):
                for u in range(16):
                    sl = pl.ds(j * 256 + u * 16, 16)
                    ots1[_par][sl] = plsc.load_gather(ins1[_par], [m1v[sl]])
                return carry

            lax.fori_loop(0, 56, c1, 0, unroll=False)
            ot1_cp(t, par).start()
        ot1_cp(12, 0).wait()
        ot1_cp(13, 1).wait()

        ins2 = (in2a, in2b)
        ots2 = (ot2a, ot2b)

        def in2_cp(t, par):
            p = jnp.minimum(wid + 32 * t, 783)
            return pltpu.make_async_copy(
                v2h.at[pl.ds(p * 4096, 4096)], ins2[par], sin[par])

        def ot2_cp(t, par):
            p = jnp.minimum(wid + 32 * t, 783)
            return pltpu.make_async_copy(
                ots2[par], o2h.at[pl.ds(p * 4096, 4096)], sot[par])

        in2_cp(0, 0).start()
        for t in range(25):
            par = t % 2
            in2_cp(t, par).wait()
            if t + 1 < 25:
                in2_cp(t + 1, 1 - par).start()
            if t >= 2:
                ot2_cp(t - 2, par).wait()

            def c2(j, carry, _par=par):
                for u in range(32):
                    sl = pl.ds(j * 512 + u * 16, 16)
                    ots2[_par][sl] = plsc.load_gather(ins2[_par], [m2v[sl]])
                return carry

            lax.fori_loop(0, _B, c2, 0, unroll=False)
            ot2_cp(t, par).start()
        ot2_cp(23, 1).wait()
        ot2_cp(24, 0).wait()

    return k(v1, v2, map1, map2)


def kernel(feat_layer1, feat_layer2, feat_layer3,
           idx_block1_layer1, idx_block1_layer2,
           idx_block2_layer2, idx_block2_layer3):
    p2 = feat_layer2.transpose(2, 3, 0, 1)
    p3 = feat_layer3.transpose(2, 3, 0, 1)

    v1 = (feat_layer1.reshape(8, 2, 128, 56, 7, 8)
          .transpose(0, 3, 4, 1, 5, 2).reshape(8 * 56 * 14336))
    v2 = (feat_layer2.reshape(8, 4, 128, 28, 28)
          .transpose(3, 4, 1, 0, 2).reshape(28 * 28 * 4096))

    k1 = jnp.arange(56 * 256, dtype=jnp.int32)
    s1 = idx_block1_layer1[k1 & 255]
    map1 = ((k1 >> 11) * 2048 + (s1 >> 7) * 1024
            + ((k1 >> 8) & 7) * 128 + (s1 & 127))
    k2 = jnp.arange(_B * 512, dtype=jnp.int32)
    s2 = idx_block2_layer2[k2 & 511]
    map2 = (s2 >> 7) * 1024 + (k2 >> 9) * 128 + (s2 & 127)

    o1v, o2v = _sc_skip_fills(v1, v2, map1, map2)
    g2 = _gather2_only(p2, idx_block1_layer2)
    g3 = _gather3(p3, idx_block2_layer3)

    o1p = _merge1(o1v.reshape(8, 56, 56, 256))
    o2p = _merge2(o2v.reshape(28, 28, 8, 512))

    o1n = _upsample1(g2, o1p)
    o2n = _upsample2(g3, o2p)

    block1 = o1n.transpose(0, 3, 1, 2)
    block2 = o2n.transpose(2, 3, 0, 1)
    return (block1, block2)


# speedup vs baseline: 1.5911x; 1.5911x over previous
import functools

import numpy as np
import jax
import jax.numpy as jnp
from jax import lax
from jax.experimental import pallas as pl
from jax.experimental.pallas import tpu as pltpu
from jax.experimental.pallas import tpu_sc as plsc

_B = 8


def _interp_mat(h):
    ys = np.linspace(0.0, h - 1, 2 * h, dtype=np.float32)
    y0 = np.floor(ys).astype(np.int32)
    y1 = np.minimum(y0 + 1, h - 1)
    wy = (ys - y0).astype(np.float32)
    m = np.zeros((2 * h, h), dtype=np.float32)
    rows = np.arange(2 * h)
    m[rows, y0] += 1.0 - wy
    m[rows, y1] += wy
    return m


def _lane_gather(x2, idx, cin, cout):
    r = x2.shape[0]
    idxb = jnp.broadcast_to((idx & 127)[None, :], (r, cout))
    hi = idx >> 7
    acc = jnp.take_along_axis(x2[:, 0:128], idxb, axis=1)
    for h in range(1, cin // 128):
        g = jnp.take_along_axis(x2[:, h * 128:(h + 1) * 128], idxb, axis=1)
        sel = jnp.broadcast_to((hi == h)[None, :], (r, cout))
        acc = jnp.where(sel, g, acc)
    return acc


def _gather1(p1, i11):
    def body(x_ref, i_ref, o_ref):
        x2 = x_ref[...].reshape(56 * 56, 256)
        g = _lane_gather(x2, i_ref[...], 256, 256)
        o_ref[...] = g.reshape(1, 56, 56, 256)

    return pl.pallas_call(
        body,
        grid=(_B,),
        in_specs=[
            pl.BlockSpec((1, 56, 56, 256), lambda b: (b, 0, 0, 0)),
            pl.BlockSpec((256,), lambda b: (0,)),
        ],
        out_specs=pl.BlockSpec((1, 56, 56, 256), lambda b: (b, 0, 0, 0)),
        out_shape=jax.ShapeDtypeStruct((_B, 56, 56, 768), jnp.float32),
    )(p1, i11)


def _gather2_only(p2, i12):
    def body(x_ref, i_ref, g_ref):
        x2 = x_ref[...].reshape(28 * 8, 512)
        g_ref[...] = _lane_gather(x2, i_ref[...], 512, 512).reshape(1, 28, 8, 512)

    return pl.pallas_call(
        body,
        grid=(28,),
        in_specs=[
            pl.BlockSpec((1, 28, 8, 512), lambda h: (h, 0, 0, 0)),
            pl.BlockSpec((512,), lambda h: (0,)),
        ],
        out_specs=pl.BlockSpec((1, 28, 8, 512), lambda h: (h, 0, 0, 0)),
        out_shape=jax.ShapeDtypeStruct((28, 28, _B, 512), jnp.float32),
    )(p2, i12)


def _gather3(p3, i23):
    def body(x_ref, i_ref, o_ref):
        x2 = x_ref[...].reshape(14 * 8, 1024)
        g = _lane_gather(x2, i_ref[...], 1024, 512)
        o_ref[...] = g.reshape(1, 14, 8, 512)

    return pl.pallas_call(
        body,
        grid=(14,),
        in_specs=[
            pl.BlockSpec((1, 14, 8, 1024), lambda h: (h, 0, 0, 0)),
            pl.BlockSpec((512,), lambda h: (0,)),
        ],
        out_specs=pl.BlockSpec((1, 14, 8, 512), lambda h: (h, 0, 0, 0)),
        out_shape=jax.ShapeDtypeStruct((14, 14, _B, 512), jnp.float32),
    )(p3, i23)


def _merge2(o2c):
    def body(x_ref, o_ref):
        o_ref[...] = x_ref[...]

    return pl.pallas_call(
        body,
        grid=(28,),
        in_specs=[pl.BlockSpec((1, 28, 8, 512), lambda h: (h, 0, 0, 0))],
        out_specs=pl.BlockSpec((1, 28, 8, 512), lambda h: (h, 0, 0, 0)),
        out_shape=jax.ShapeDtypeStruct((28, 28, _B, 1024), jnp.float32),
    )(o2c)


def _upsample1(g2, o1n_partial):
    m2 = jnp.asarray(_interp_mat(28))

    def body(o_alias_ref, top_ref, bot_ref, m_ref, o_ref):
        del o_alias_ref
        i = pl.program_id(1)
        wy = (((i * 27) % 55).astype(jnp.float32) / 55.0).astype(jnp.float32)
        u = top_ref[0] * (1.0 - wy) + bot_ref[0] * wy
        v = jax.lax.dot_general(m_ref[...], u, (((1,), (0,)), ((), ())),
                                preferred_element_type=jnp.float32)
        o_ref[...] = v.transpose(1, 0, 2).reshape(_B, 1, 56, 256)

    return pl.pallas_call(
        body,
        grid=(2, 56),
        in_specs=[
            pl.BlockSpec(memory_space=pl.ANY),
            pl.BlockSpec((1, 28, 8, 256), lambda cq, i: ((i * 27) // 55, 0, 0, cq)),
            pl.BlockSpec((1, 28, 8, 256),
                         lambda cq, i: (jnp.minimum((i * 27) // 55 + 1, 27), 0, 0, cq)),
            pl.BlockSpec((56, 28), lambda cq, i: (0, 0)),
        ],
        out_specs=pl.BlockSpec((_B, 1, 56, 256), lambda cq, i: (0, i, 0, 1 + cq)),
        out_shape=jax.ShapeDtypeStruct((_B, 56, 56, 768), jnp.float32),
        input_output_aliases={0: 0},
    )(o1n_partial, g2, g2, m2)


def _upsample2(g3, o2n_partial):
    m3 = jnp.asarray(_interp_mat(14))

    def body(o_alias_ref, top_ref, bot_ref, m_ref, o_ref):
        del o_alias_ref
        i = pl.program_id(0)
        wy = (((i * 13) % 27).astype(jnp.float32) / 27.0).astype(jnp.float32)
        u = top_ref[0] * (1.0 - wy) + bot_ref[0] * wy
        v = jax.lax.dot_general(m_ref[...], u, (((1,), (0,)), ((), ())),
                                preferred_element_type=jnp.float32)
        o_ref[...] = v.reshape(1, 28, _B, 512)

    return pl.pallas_call(
        body,
        grid=(28,),
        in_specs=[
            pl.BlockSpec(memory_space=pl.ANY),
            pl.BlockSpec((1, 14, 8, 512), lambda i: ((i * 13) // 27, 0, 0, 0)),
            pl.BlockSpec((1, 14, 8, 512),
                         lambda i: (jnp.minimum((i * 13) // 27 + 1, 13), 0, 0, 0)),
            pl.BlockSpec((28, 14), lambda i: (0, 0)),
        ],
        out_specs=pl.BlockSpec((1, 28, _B, 512), lambda i: (i, 0, 0, 1)),
        out_shape=jax.ShapeDtypeStruct((28, 28, _B, 1024), jnp.float32),
        input_output_aliases={0: 0},
    )(o2n_partial, g3, g3, m3)


def _sc_skip_fill2(v2, map2):
    mesh = plsc.VectorSubcoreMesh(core_axis_name="c", subcore_axis_name="s")
    out_type = jax.ShapeDtypeStruct((784 * 4096,), jnp.float32)
    scratch_types = [
        pltpu.VMEM((4096,), jnp.int32),
        pltpu.VMEM((4096,), jnp.float32),
        pltpu.VMEM((4096,), jnp.float32),
    ]

    @functools.partial(pl.kernel, out_type=out_type, mesh=mesh,
                       scratch_types=scratch_types,
                       compiler_params=pltpu.CompilerParams(
                           needs_layout_passes=False))
    def k(v2h, m2h, o2h, m2v, buf2, obuf2):
        wid = lax.axis_index("s") * 2 + lax.axis_index("c")
        pltpu.sync_copy(m2h, m2v)

        start = wid * 24 + jnp.minimum(wid, 16)
        cnt = 24 + (wid < 16).astype(jnp.int32)

        def pos_body(p, carry):
            pltpu.sync_copy(v2h.at[pl.ds(p * 4096, 4096)], buf2)

            def c2(j, carry2):
                for u in range(32):
                    sl = pl.ds(j * 512 + u * 16, 16)
                    obuf2[sl] = plsc.load_gather(buf2, [m2v[sl]])
                return carry2

            lax.fori_loop(0, _B, c2, 0, unroll=False)
            pltpu.sync_copy(obuf2, o2h.at[pl.ds(p * 4096, 4096)])
            return carry

        lax.fori_loop(start, start + cnt, pos_body, 0)

    return k(v2, map2)


def kernel(feat_layer1, feat_layer2, feat_layer3,
           idx_block1_layer1, idx_block1_layer2,
           idx_block2_layer2, idx_block2_layer3):
    p1 = feat_layer1.transpose(0, 2, 3, 1)
    p2 = feat_layer2.transpose(2, 3, 0, 1)
    p3 = feat_layer3.transpose(2, 3, 0, 1)

    v2 = (feat_layer2.reshape(8, 4, 128, 28, 28)
          .transpose(3, 4, 1, 0, 2).reshape(28 * 28 * 4096))

    k2 = jnp.arange(_B * 512, dtype=jnp.int32)
    s2 = idx_block2_layer2[k2 & 511]
    map2 = (s2 >> 7) * 1024 + (k2 >> 9) * 128 + (s2 & 127)

    o2v = _sc_skip_fill2(v2, map2)
    o1p = _gather1(p1, idx_block1_layer1)
    g2 = _gather2_only(p2, idx_block1_layer2)
    g3 = _gather3(p3, idx_block2_layer3)

    o2p = _merge2(o2v.reshape(28, 28, 8, 512))

    o1n = _upsample1(g2, o1p)
    o2n = _upsample2(g3, o2p)

    block1 = o1n.transpose(0, 3, 1, 2)
    block2 = o2n.transpose(2, 3, 0, 1)
    return (block1, block2)


# speedup vs baseline: 1.8054x; 1.1347x over previous
import functools

import numpy as np
import jax
import jax.numpy as jnp
from jax import lax
from jax.experimental import pallas as pl
from jax.experimental.pallas import tpu as pltpu
from jax.experimental.pallas import tpu_sc as plsc

_B = 8


def _interp_mat(h):
    ys = np.linspace(0.0, h - 1, 2 * h, dtype=np.float32)
    y0 = np.floor(ys).astype(np.int32)
    y1 = np.minimum(y0 + 1, h - 1)
    wy = (ys - y0).astype(np.float32)
    m = np.zeros((2 * h, h), dtype=np.float32)
    rows = np.arange(2 * h)
    m[rows, y0] += 1.0 - wy
    m[rows, y1] += wy
    return m


def _lane_gather(x2, idx, cin, cout):
    r = x2.shape[0]
    idxb = jnp.broadcast_to((idx & 127)[None, :], (r, cout))
    hi = idx >> 7
    acc = jnp.take_along_axis(x2[:, 0:128], idxb, axis=1)
    for h in range(1, cin // 128):
        g = jnp.take_along_axis(x2[:, h * 128:(h + 1) * 128], idxb, axis=1)
        sel = jnp.broadcast_to((hi == h)[None, :], (r, cout))
        acc = jnp.where(sel, g, acc)
    return acc


def _gather1(p1, i11):
    def body(x_ref, i_ref, o_ref):
        x2 = x_ref[...].reshape(56 * 56, 256)
        g = _lane_gather(x2, i_ref[...], 256, 256)
        o_ref[...] = g.reshape(1, 56, 56, 256)

    return pl.pallas_call(
        body,
        grid=(_B,),
        in_specs=[
            pl.BlockSpec((1, 56, 56, 256), lambda b: (b, 0, 0, 0)),
            pl.BlockSpec((256,), lambda b: (0,)),
        ],
        out_specs=pl.BlockSpec((1, 56, 56, 256), lambda b: (b, 0, 0, 0)),
        out_shape=jax.ShapeDtypeStruct((_B, 56, 56, 768), jnp.float32),
    )(p1, i11)


def _gather2_only(p2, i12):
    def body(x_ref, i_ref, g_ref):
        x2 = x_ref[...].reshape(28 * 8, 512)
        g_ref[...] = _lane_gather(x2, i_ref[...], 512, 512).reshape(1, 28, 8, 512)

    return pl.pallas_call(
        body,
        grid=(28,),
        in_specs=[
            pl.BlockSpec((1, 28, 8, 512), lambda h: (h, 0, 0, 0)),
            pl.BlockSpec((512,), lambda h: (0,)),
        ],
        out_specs=pl.BlockSpec((1, 28, 8, 512), lambda h: (h, 0, 0, 0)),
        out_shape=jax.ShapeDtypeStruct((28, 28, _B, 512), jnp.float32),
    )(p2, i12)


def _gather3(p3, i23):
    def body(x_ref, i_ref, o_ref):
        x2 = x_ref[...].reshape(14 * 8, 1024)
        g = _lane_gather(x2, i_ref[...], 1024, 512)
        o_ref[...] = g.reshape(1, 14, 8, 512)

    return pl.pallas_call(
        body,
        grid=(14,),
        in_specs=[
            pl.BlockSpec((1, 14, 8, 1024), lambda h: (h, 0, 0, 0)),
            pl.BlockSpec((512,), lambda h: (0,)),
        ],
        out_specs=pl.BlockSpec((1, 14, 8, 512), lambda h: (h, 0, 0, 0)),
        out_shape=jax.ShapeDtypeStruct((14, 14, _B, 512), jnp.float32),
    )(p3, i23)


def _upsample1(g2, o1n_partial):
    m2 = jnp.asarray(_interp_mat(28))

    def body(o_alias_ref, top_ref, bot_ref, m_ref, o_ref):
        del o_alias_ref
        i = pl.program_id(1)
        wy = (((i * 27) % 55).astype(jnp.float32) / 55.0).astype(jnp.float32)
        u = top_ref[0] * (1.0 - wy) + bot_ref[0] * wy
        v = jax.lax.dot_general(m_ref[...], u, (((1,), (0,)), ((), ())),
                                preferred_element_type=jnp.float32)
        o_ref[...] = v.transpose(1, 0, 2).reshape(_B, 1, 56, 256)

    return pl.pallas_call(
        body,
        grid=(2, 56),
        in_specs=[
            pl.BlockSpec(memory_space=pl.ANY),
            pl.BlockSpec((1, 28, 8, 256), lambda cq, i: ((i * 27) // 55, 0, 0, cq)),
            pl.BlockSpec((1, 28, 8, 256),
                         lambda cq, i: (jnp.minimum((i * 27) // 55 + 1, 27), 0, 0, cq)),
            pl.BlockSpec((56, 28), lambda cq, i: (0, 0)),
        ],
        out_specs=pl.BlockSpec((_B, 1, 56, 256), lambda cq, i: (0, i, 0, 1 + cq)),
        out_shape=jax.ShapeDtypeStruct((_B, 56, 56, 768), jnp.float32),
        input_output_aliases={0: 0},
    )(o1n_partial, g2, g2, m2)


def _upsample2(g3, o2n_partial):
    m3 = jnp.asarray(_interp_mat(14))

    def body(o_alias_ref, top_ref, bot_ref, m_ref, o_ref):
        del o_alias_ref
        i = pl.program_id(0)
        wy = (((i * 13) % 27).astype(jnp.float32) / 27.0).astype(jnp.float32)
        u = top_ref[0] * (1.0 - wy) + bot_ref[0] * wy
        v = jax.lax.dot_general(m_ref[...], u, (((1,), (0,)), ((), ())),
                                preferred_element_type=jnp.float32)
        o_ref[...] = v.reshape(1, 28, _B, 512)

    return pl.pallas_call(
        body,
        grid=(28,),
        in_specs=[
            pl.BlockSpec(memory_space=pl.ANY),
            pl.BlockSpec((1, 14, 8, 512), lambda i: ((i * 13) // 27, 0, 0, 0)),
            pl.BlockSpec((1, 14, 8, 512),
                         lambda i: (jnp.minimum((i * 13) // 27 + 1, 13), 0, 0, 0)),
            pl.BlockSpec((28, 14), lambda i: (0, 0)),
        ],
        out_specs=pl.BlockSpec((1, 28, _B, 512), lambda i: (i, 0, 0, 1)),
        out_shape=jax.ShapeDtypeStruct((28, 28, _B, 1024), jnp.float32),
        input_output_aliases={0: 0},
    )(o2n_partial, g3, g3, m3)


def _sc_skip_fill2(v2, map2):
    mesh = plsc.VectorSubcoreMesh(core_axis_name="c", subcore_axis_name="s")
    out_type = jax.ShapeDtypeStruct((784 * 8192,), jnp.float32)
    scratch_types = [
        pltpu.VMEM((4096,), jnp.int32),
        pltpu.VMEM((4096,), jnp.float32),
        pltpu.VMEM((4096,), jnp.float32),
    ]

    @functools.partial(pl.kernel, out_type=out_type, mesh=mesh,
                       scratch_types=scratch_types,
                       compiler_params=pltpu.CompilerParams(
                           needs_layout_passes=False))
    def k(v2h, m2h, o2h, m2v, buf2, obuf2):
        wid = lax.axis_index("s") * 2 + lax.axis_index("c")
        pltpu.sync_copy(m2h, m2v)

        start = wid * 24 + jnp.minimum(wid, 16)
        cnt = 24 + (wid < 16).astype(jnp.int32)

        def pos_body(p, carry):
            pltpu.sync_copy(v2h.at[pl.ds(p * 4096, 4096)], buf2)

            def c2(j, carry2):
                for u in range(32):
                    sl = pl.ds(j * 512 + u * 16, 16)
                    obuf2[sl] = plsc.load_gather(buf2, [m2v[sl]])
                return carry2

            lax.fori_loop(0, _B, c2, 0, unroll=False)
            pltpu.sync_copy(obuf2, o2h.at[pl.ds(p * 8192, 4096)])
            return carry

        lax.fori_loop(start, start + cnt, pos_body, 0)

    return k(v2, map2)


def kernel(feat_layer1, feat_layer2, feat_layer3,
           idx_block1_layer1, idx_block1_layer2,
           idx_block2_layer2, idx_block2_layer3):
    p1 = feat_layer1.transpose(0, 2, 3, 1)
    p2 = feat_layer2.transpose(2, 3, 0, 1)
    p3 = feat_layer3.transpose(2, 3, 0, 1)

    v2 = (feat_layer2.reshape(8, 4, 128, 28, 28)
          .transpose(3, 4, 1, 0, 2).reshape(28 * 28 * 4096))

    k2 = jnp.arange(_B * 512, dtype=jnp.int32)
    s2 = idx_block2_layer2[(k2 >> 10) * 128 + (k2 & 127)]
    map2 = (s2 >> 7) * 1024 + (k2 & 0x380) + (s2 & 127)

    o2v = _sc_skip_fill2(v2, map2)
    o1p = _gather1(p1, idx_block1_layer1)
    g2 = _gather2_only(p2, idx_block1_layer2)
    g3 = _gather3(p3, idx_block2_layer3)

    o2p = (o2v.reshape(28, 28, 8, 8, 128).transpose(0, 1, 3, 2, 4)
           .reshape(28, 28, 8, 1024))

    o1n = _upsample1(g2, o1p)
    o2n = _upsample2(g3, o2p)

    block1 = o1n.transpose(0, 3, 1, 2)
    block2 = o2n.transpose(2, 3, 0, 1)
    return (block1, block2)
